# baseline (speedup 1.0000x reference)
import jax
import jax.numpy as jnp
from jax import lax
from jax.experimental import pallas as pl
from jax.experimental.pallas import tpu as pltpu
from jax.experimental.pallas import tpu_sc as plsc

_NUM_EMBEDDINGS = 8192
_EMBEDDING_DIM = 256
_COMMITMENT_COST = 0.25
_M = 16384
_BM = 128
_NSTEPS = _M // _BM
_CB = (0, 2736, 5472, _NUM_EMBEDDINGS)

_NC = 2
_NS = 16
_NW = _NC * _NS
_B_PER_W = _M // _NW
_CHUNK = 128
_NCHUNK = _B_PER_W // _CHUNK


def _rowsum_sq(v):
    t = v * v
    s = t[:, :128] + t[:, 128:]
    a = s[:, 0:8]
    for g in range(1, 16):
        a = a + s[:, 8 * g:8 * g + 8]
    a = a[:, 0:4] + a[:, 4:8]
    a = a[:, 0:2] + a[:, 2:4]
    a = a[:, 0:1] + a[:, 1:2]
    return a


def _bf16_round(v):
    return v.astype(jnp.bfloat16).astype(jnp.float32)


def _es_body(e_ref, es_ref):
    es_ref[...] = _rowsum_sq(e_ref[...])


_es_call = pl.pallas_call(
    _es_body,
    grid=(8,),
    in_specs=[pl.BlockSpec((_NUM_EMBEDDINGS // 8, _EMBEDDING_DIM),
                           lambda i: (i, 0))],
    out_specs=pl.BlockSpec((_NUM_EMBEDDINGS // 8, 1), lambda i: (i, 0)),
    out_shape=jax.ShapeDtypeStruct((_NUM_EMBEDDINGS, 1), jnp.float32),
)


def _vq_tc_body(x_ref, e_ref, es_ref, idx_ref, enc_ref, loss_ref, perp_ref,
                iota3_ref, idxv_ref, counts_ref, lacc_ref):
    i = pl.program_id(0)

    @pl.when(i == 0)
    def _init():
        iota1 = lax.broadcasted_iota(jnp.int32, (1, _NUM_EMBEDDINGS), 1)
        cid = ((iota1 >= _CB[1]).astype(jnp.int32)
               + (iota1 >= _CB[2]).astype(jnp.int32))
        iota3_ref[...] = (iota1 + cid * _NUM_EMBEDDINGS).astype(jnp.float32)
        counts_ref[...] = jnp.zeros_like(counts_ref)
        lacc_ref[...] = jnp.zeros_like(lacc_ref)

    x = x_ref[...]
    xs = _rowsum_sq(x)
    es = es_ref[...]
    mm2 = lax.dot_general(x * jnp.float32(-2.0), e_ref[...],
                          (((1,), (1,)), ((), ())),
                          preferred_element_type=jnp.float32)
    d = (xs + es) + mm2

    def _fold(t0, t1):
        n = t1 - t0
        if n == 1:
            return d[:, t0 * 128:(t0 + 1) * 128]
        h = 1 << (n.bit_length() - 1)
        if h == n:
            h //= 2
        return jnp.minimum(_fold(t0, t0 + h), _fold(t0 + h, t1))
    def _lmin(v):
        return jnp.min(v, axis=1, keepdims=True)
    def _pmin(lo, width):
        return jnp.min(d[:, lo:lo + width], axis=1, keepdims=True)
    m0 = jnp.minimum(_lmin(_fold(0, 21)), _pmin(2688, 48))
    m1 = jnp.minimum(jnp.minimum(_pmin(2736, 80), _lmin(_fold(22, 42))),
                     _pmin(5376, 96))
    m2 = jnp.minimum(_pmin(5472, 32), _lmin(_fold(43, 64)))
    a_v = _bf16_round(m0)
    a_d = m0
    a_c = jnp.zeros_like(m0, dtype=jnp.int32)
    take1 = m1 < a_v
    a_v = jnp.where(take1, _bf16_round(m1), a_v)
    a_d = jnp.where(take1, m1, a_d)
    a_c = jnp.where(take1, 1, a_c)
    take2 = m2 < a_v
    a_d = jnp.where(take2, m2, a_d)
    a_c = jnp.where(take2, 2, a_c)

    cand = jnp.where(d == a_d, iota3_ref[...], jnp.float32(1e9))
    v = jnp.min(cand, axis=1, keepdims=True)
    idxv_ref[...] = v.astype(jnp.int32) - a_c * _NUM_EMBEDDINGS

    @pl.when(jnp.any(idxv_ref[...] < 0))
    def _slow():
        iota_p = lax.broadcasted_iota(jnp.int32, (_BM, _NUM_EMBEDDINGS), 1)
        cid = ((iota_p >= _CB[1]).astype(jnp.int32)
               + (iota_p >= _CB[2]).astype(jnp.int32))
        sel = (cid == a_c) & (d == a_d)
        idxv_ref[...] = jnp.min(
            jnp.where(sel, iota_p, _NUM_EMBEDDINGS), axis=1, keepdims=True)

    idx = idxv_ref[...][:, 0]
    iota = lax.broadcasted_iota(jnp.int32, (_BM, _NUM_EMBEDDINGS), 1)
    enc = (iota == idx[:, None]).astype(jnp.float32)
    idx_ref[...] = jnp.transpose(idxv_ref[...])[None]
    enc_ref[...] = enc
    counts_ref[...] += jnp.sum(enc, axis=0, keepdims=True)
    lacc_ref[...] += a_d

    @pl.when(i == _NSTEPS - 1)
    def _fin():
        loss_ref[...] = jnp.broadcast_to(
            jnp.sum(lacc_ref[...]) * (_COMMITMENT_COST / (_M * _EMBEDDING_DIM)),
            (1, 1))
        p = counts_ref[...] * (1.0 / _M)
        ent = -jnp.sum(p * jnp.log(p + 1e-10))
        perp_ref[...] = jnp.broadcast_to(jnp.exp(ent), (1, 1))


_vq_tc = pl.pallas_call(
    _vq_tc_body,
    grid=(_NSTEPS,),
    in_specs=[
        pl.BlockSpec((_BM, _EMBEDDING_DIM), lambda i: (i, 0)),
        pl.BlockSpec((_NUM_EMBEDDINGS, _EMBEDDING_DIM), lambda i: (0, 0)),
        pl.BlockSpec((1, _NUM_EMBEDDINGS), lambda i: (0, 0)),
    ],
    out_specs=[
        pl.BlockSpec((1, 1, _BM), lambda i: (i, 0, 0)),
        pl.BlockSpec((_BM, _NUM_EMBEDDINGS), lambda i: (i, 0)),
        pl.BlockSpec((1, 1), lambda i: (0, 0)),
        pl.BlockSpec((1, 1), lambda i: (0, 0)),
    ],
    out_shape=[
        jax.ShapeDtypeStruct((_NSTEPS, 1, _BM), jnp.int32),
        jax.ShapeDtypeStruct((_M, _NUM_EMBEDDINGS), jnp.float32),
        jax.ShapeDtypeStruct((1, 1), jnp.float32),
        jax.ShapeDtypeStruct((1, 1), jnp.float32),
    ],
    scratch_shapes=[
        pltpu.VMEM((1, _NUM_EMBEDDINGS), jnp.float32),
        pltpu.VMEM((_BM, 1), jnp.int32),
        pltpu.VMEM((1, _NUM_EMBEDDINGS), jnp.float32),
        pltpu.VMEM((_BM, 1), jnp.float32),
    ],
)


def _sc_gather_body(e_hbm, idx_hbm, out_hbm, idx_v, rows_v, sem):
    wid = lax.axis_index("s") * _NC + lax.axis_index("c")
    base = wid * _B_PER_W
    for c in range(_NCHUNK):
        off = base + c * _CHUNK
        pltpu.sync_copy(idx_hbm.at[pl.ds(off, _CHUNK)], idx_v)
        pltpu.async_copy(e_hbm.at[idx_v], rows_v, sem).wait()
        pltpu.sync_copy(rows_v, out_hbm.at[pl.ds(off, _CHUNK)])


def _sc_gather(embedding_weight, idx_flat):
    call = pl.kernel(
        _sc_gather_body,
        out_type=jax.ShapeDtypeStruct((_M, _EMBEDDING_DIM), jnp.float32),
        mesh=plsc.VectorSubcoreMesh(core_axis_name="c", subcore_axis_name="s",
                                    num_cores=_NC, num_subcores=_NS),
        scratch_types=[
            pltpu.VMEM((_CHUNK,), jnp.int32),
            pltpu.VMEM((_CHUNK, _EMBEDDING_DIM), jnp.float32),
            pltpu.SemaphoreType.DMA,
        ],
    )
    return call(embedding_weight, idx_flat)


def kernel(inputs, embedding_weight):
    input_shape = inputs.shape
    flat_input = inputs.reshape(-1, _EMBEDDING_DIM)
    es_row = _es_call(embedding_weight).reshape(1, _NUM_EMBEDDINGS)
    idx2d, encodings, loss, perp = _vq_tc(flat_input, embedding_weight, es_row)
    idx_flat = idx2d.reshape(_M)
    quantized = _sc_gather(embedding_weight, idx_flat)
    quantized_st = quantized.reshape(input_shape)
    quantized_ind = idx_flat.reshape(input_shape[:-1])
    return (loss.reshape(()), quantized_st, quantized_ind,
            perp.reshape(()), encodings)


# speedup vs baseline: 1.3483x; 1.3483x over previous
import jax
import jax.numpy as jnp
from jax import lax
from jax.experimental import pallas as pl
from jax.experimental.pallas import tpu as pltpu
from jax.experimental.pallas import tpu_sc as plsc

_NUM_EMBEDDINGS = 8192
_EMBEDDING_DIM = 256
_COMMITMENT_COST = 0.25
_M = 16384
_BM = 256
_NSTEPS = _M // _BM
_CB = (0, 2736, 5472, _NUM_EMBEDDINGS)

_NC = 2
_NS = 16
_NW = _NC * _NS
_B_PER_W = _M // _NW
_CHUNK = 128
_NCHUNK = _B_PER_W // _CHUNK


def _rowsum_sq(v):
    t = v * v
    s = t[:, :128] + t[:, 128:]
    a = s[:, 0:8]
    for g in range(1, 16):
        a = a + s[:, 8 * g:8 * g + 8]
    a = a[:, 0:4] + a[:, 4:8]
    a = a[:, 0:2] + a[:, 2:4]
    a = a[:, 0:1] + a[:, 1:2]
    return a


def _bf16_round(v):
    return v.astype(jnp.bfloat16).astype(jnp.float32)


def _es_body(e_ref, es_ref):
    es_ref[...] = _rowsum_sq(e_ref[...])


_es_call = pl.pallas_call(
    _es_body,
    grid=(8,),
    in_specs=[pl.BlockSpec((_NUM_EMBEDDINGS // 8, _EMBEDDING_DIM),
                           lambda i: (i, 0))],
    out_specs=pl.BlockSpec((_NUM_EMBEDDINGS // 8, 1), lambda i: (i, 0)),
    out_shape=jax.ShapeDtypeStruct((_NUM_EMBEDDINGS, 1), jnp.float32),
)


def _vq_tc_body(x_ref, e_ref, es_ref, idx_ref, enc_ref, loss_ref, perp_ref,
                iota3_ref, idxv_ref, counts_ref, lacc_ref):
    i = pl.program_id(0)

    @pl.when(i == 0)
    def _init():
        iota1 = lax.broadcasted_iota(jnp.int32, (1, _NUM_EMBEDDINGS), 1)
        cid = ((iota1 >= _CB[1]).astype(jnp.int32)
               + (iota1 >= _CB[2]).astype(jnp.int32))
        iota3_ref[...] = (iota1 + cid * _NUM_EMBEDDINGS).astype(jnp.float32)
        counts_ref[...] = jnp.zeros_like(counts_ref)
        lacc_ref[...] = jnp.zeros_like(lacc_ref)

    x = x_ref[...]
    xs = _rowsum_sq(x)
    es = es_ref[...]
    mm2 = lax.dot_general(x * jnp.float32(-2.0), e_ref[...],
                          (((1,), (1,)), ((), ())),
                          preferred_element_type=jnp.float32)
    d = (xs + es) + mm2

    def _fold(t0, t1):
        n = t1 - t0
        if n == 1:
            return d[:, t0 * 128:(t0 + 1) * 128]
        h = 1 << (n.bit_length() - 1)
        if h == n:
            h //= 2
        return jnp.minimum(_fold(t0, t0 + h), _fold(t0 + h, t1))
    def _lmin(v):
        return jnp.min(v, axis=1, keepdims=True)
    def _pmin(lo, width):
        return jnp.min(d[:, lo:lo + width], axis=1, keepdims=True)
    m0 = jnp.minimum(_lmin(_fold(0, 21)), _pmin(2688, 48))
    m1 = jnp.minimum(jnp.minimum(_pmin(2736, 80), _lmin(_fold(22, 42))),
                     _pmin(5376, 96))
    m2 = jnp.minimum(_pmin(5472, 32), _lmin(_fold(43, 64)))
    a_v = _bf16_round(m0)
    a_d = m0
    a_c = jnp.zeros_like(m0, dtype=jnp.int32)
    take1 = m1 < a_v
    a_v = jnp.where(take1, _bf16_round(m1), a_v)
    a_d = jnp.where(take1, m1, a_d)
    a_c = jnp.where(take1, 1, a_c)
    take2 = m2 < a_v
    a_d = jnp.where(take2, m2, a_d)
    a_c = jnp.where(take2, 2, a_c)

    cand = jnp.where(d == a_d, iota3_ref[...], jnp.float32(1e9))
    v = jnp.min(cand, axis=1, keepdims=True)
    idxv_ref[...] = v.astype(jnp.int32) - a_c * _NUM_EMBEDDINGS

    @pl.when(jnp.any(idxv_ref[...] < 0))
    def _slow():
        iota_p = lax.broadcasted_iota(jnp.int32, (_BM, _NUM_EMBEDDINGS), 1)
        cid = ((iota_p >= _CB[1]).astype(jnp.int32)
               + (iota_p >= _CB[2]).astype(jnp.int32))
        sel = (cid == a_c) & (d == a_d)
        idxv_ref[...] = jnp.min(
            jnp.where(sel, iota_p, _NUM_EMBEDDINGS), axis=1, keepdims=True)

    idx = idxv_ref[...][:, 0]
    iota = lax.broadcasted_iota(jnp.int32, (_BM, _NUM_EMBEDDINGS), 1)
    enc = (iota == idx[:, None]).astype(jnp.float32)
    idx_ref[...] = jnp.transpose(idxv_ref[...])[None]
    enc_ref[...] = enc
    counts_ref[...] += jnp.sum(enc, axis=0, keepdims=True)
    lacc_ref[...] += a_d

    @pl.when(i == _NSTEPS - 1)
    def _fin():
        loss_ref[...] = jnp.broadcast_to(
            jnp.sum(lacc_ref[...]) * (_COMMITMENT_COST / (_M * _EMBEDDING_DIM)),
            (1, 1))
        p = counts_ref[...] * (1.0 / _M)
        ent = -jnp.sum(p * jnp.log(p + 1e-10))
        perp_ref[...] = jnp.broadcast_to(jnp.exp(ent), (1, 1))


_vq_tc = pl.pallas_call(
    _vq_tc_body,
    grid=(_NSTEPS,),
    in_specs=[
        pl.BlockSpec((_BM, _EMBEDDING_DIM), lambda i: (i, 0)),
        pl.BlockSpec((_NUM_EMBEDDINGS, _EMBEDDING_DIM), lambda i: (0, 0)),
        pl.BlockSpec((1, _NUM_EMBEDDINGS), lambda i: (0, 0)),
    ],
    out_specs=[
        pl.BlockSpec((1, 1, _BM), lambda i: (i, 0, 0)),
        pl.BlockSpec((_BM, _NUM_EMBEDDINGS), lambda i: (i, 0)),
        pl.BlockSpec((1, 1), lambda i: (0, 0)),
        pl.BlockSpec((1, 1), lambda i: (0, 0)),
    ],
    out_shape=[
        jax.ShapeDtypeStruct((_NSTEPS, 1, _BM), jnp.int32),
        jax.ShapeDtypeStruct((_M, _NUM_EMBEDDINGS), jnp.float32),
        jax.ShapeDtypeStruct((1, 1), jnp.float32),
        jax.ShapeDtypeStruct((1, 1), jnp.float32),
    ],
    scratch_shapes=[
        pltpu.VMEM((1, _NUM_EMBEDDINGS), jnp.float32),
        pltpu.VMEM((_BM, 1), jnp.int32),
        pltpu.VMEM((1, _NUM_EMBEDDINGS), jnp.float32),
        pltpu.VMEM((_BM, 1), jnp.float32),
    ],
)


def _sc_gather_body(e_hbm, idx_hbm, out_hbm, idx_v, rows_v, sem):
    wid = lax.axis_index("s") * _NC + lax.axis_index("c")
    base = wid * _B_PER_W
    for c in range(_NCHUNK):
        off = base + c * _CHUNK
        pltpu.sync_copy(idx_hbm.at[pl.ds(off, _CHUNK)], idx_v)
        pltpu.async_copy(e_hbm.at[idx_v], rows_v, sem).wait()
        pltpu.sync_copy(rows_v, out_hbm.at[pl.ds(off, _CHUNK)])


def _sc_gather(embedding_weight, idx_flat):
    call = pl.kernel(
        _sc_gather_body,
        out_type=jax.ShapeDtypeStruct((_M, _EMBEDDING_DIM), jnp.float32),
        mesh=plsc.VectorSubcoreMesh(core_axis_name="c", subcore_axis_name="s",
                                    num_cores=_NC, num_subcores=_NS),
        scratch_types=[
            pltpu.VMEM((_CHUNK,), jnp.int32),
            pltpu.VMEM((_CHUNK, _EMBEDDING_DIM), jnp.float32),
            pltpu.SemaphoreType.DMA,
        ],
    )
    return call(embedding_weight, idx_flat)


def kernel(inputs, embedding_weight):
    input_shape = inputs.shape
    flat_input = inputs.reshape(-1, _EMBEDDING_DIM)
    es_row = _es_call(embedding_weight).reshape(1, _NUM_EMBEDDINGS)
    idx2d, encodings, loss, perp = _vq_tc(flat_input, embedding_weight, es_row)
    idx_flat = idx2d.reshape(_M)
    quantized = _sc_gather(embedding_weight, idx_flat)
    quantized_st = quantized.reshape(input_shape)
    quantized_ind = idx_flat.reshape(input_shape[:-1])
    return (loss.reshape(()), quantized_st, quantized_ind,
            perp.reshape(()), encodings)
